# baseline (speedup 1.0000x reference)
import functools

import jax
import jax.numpy as jnp
from jax import lax
from jax.experimental import pallas as pl
from jax.experimental.pallas import tpu as pltpu
from jax.experimental.pallas import tpu_sc as plsc

NC = 2
NS = 16
NW = NC * NS
K = 128
NBUF = 4
LOOK = 2


def kernel(phone, table):
    B, S = phone.shape
    V, D = table.shape
    N = B * S
    per_w = N // NW
    n_chunks = per_w // K
    idx3 = phone.reshape(NW, n_chunks, K)

    mesh = plsc.VectorSubcoreMesh(core_axis_name="c", subcore_axis_name="s")

    @functools.partial(
        pl.kernel,
        mesh=mesh,
        out_type=jax.ShapeDtypeStruct((N, D), jnp.float32),
        compiler_params=pltpu.CompilerParams(use_tc_tiling_on_sc=False),
        scratch_types=[
            pltpu.VMEM((n_chunks, K), jnp.int32),
            pltpu.VMEM((NBUF, K, D), jnp.float32),
            pltpu.SemaphoreType.DMA((NBUF,)),
            pltpu.SemaphoreType.DMA((NBUF,)),
        ],
    )
    def emb(idx_hbm, table_hbm, out_hbm, idx_v, rows_v, gsem, ssem):
        wid = lax.axis_index("s") * NC + lax.axis_index("c")
        base = wid * per_w
        pltpu.sync_copy(idx_hbm.at[wid], idx_v)

        def out_slice(j):
            return out_hbm.at[pl.ds(base + j * K, K)]

        def fire_gather(j, b):
            pltpu.async_copy(table_hbm.at[idx_v.at[j]], rows_v.at[b], gsem.at[b])

        for i in range(LOOK):
            fire_gather(i, i % NBUF)

        def body(j, carry):
            jn = j + LOOK
            bn = lax.rem(jn, NBUF)

            @pl.when(jn < n_chunks)
            def _():
                @pl.when(jn >= NBUF)
                def _():
                    pltpu.make_async_copy(
                        rows_v.at[bn], out_slice(jn - NBUF), ssem.at[bn]
                    ).wait()

                fire_gather(jn, bn)

            b = lax.rem(j, NBUF)
            pltpu.make_async_copy(
                table_hbm.at[idx_v.at[j]], rows_v.at[b], gsem.at[b]
            ).wait()
            pltpu.async_copy(rows_v.at[b], out_slice(j), ssem.at[b])
            return carry

        lax.fori_loop(0, n_chunks, body, 0)

        for i in range(NBUF):
            j = n_chunks - NBUF + i
            pltpu.make_async_copy(
                rows_v.at[j % NBUF], out_slice(j), ssem.at[j % NBUF]
            ).wait()

    out = emb(idx3, table)
    return out.reshape(B, S, D)


# speedup vs baseline: 1.0308x; 1.0308x over previous
import functools

import jax
import jax.numpy as jnp
from jax import lax
from jax.experimental import pallas as pl
from jax.experimental.pallas import tpu as pltpu
from jax.experimental.pallas import tpu_sc as plsc

NC = 2
NS = 16
NW = NC * NS
K = 128
NBG = 2
NBS = 4


def kernel(phone, table):
    B, S = phone.shape
    V, D = table.shape
    N = B * S
    per_w = N // NW
    n_chunks = per_w // K
    idx3 = phone.reshape(NW, n_chunks, K)
    table_p = jnp.pad(table, ((0, 0), (0, 128 - D)))

    mesh = plsc.VectorSubcoreMesh(core_axis_name="c", subcore_axis_name="s")

    @functools.partial(
        pl.kernel,
        mesh=mesh,
        out_type=jax.ShapeDtypeStruct((N, D), jnp.float32),
        scratch_types=[
            pltpu.VMEM((n_chunks, K), jnp.int32),
            pltpu.VMEM((NBG, K, 128), jnp.float32),
            pltpu.VMEM((NBS, K, D), jnp.float32),
            pltpu.SemaphoreType.DMA((NBG,)),
            pltpu.SemaphoreType.DMA((NBS,)),
        ],
    )
    def emb(idx_hbm, table_hbm, out_hbm, idx_v, rows_v, cbuf, gsem, ssem):
        wid = lax.axis_index("s") * NC + lax.axis_index("c")
        base = wid * per_w
        pltpu.sync_copy(idx_hbm.at[wid], idx_v)

        def out_slice(j):
            return out_hbm.at[pl.ds(base + j * K, K)]

        def fire_gather(j, b):
            pltpu.async_copy(table_hbm.at[idx_v.at[j]], rows_v.at[b], gsem.at[b])

        fire_gather(0, 0)

        def body(j, carry):
            jn = j + 1

            @pl.when(jn < n_chunks)
            def _():
                fire_gather(jn, lax.rem(jn, NBG))

            bg = lax.rem(j, NBG)
            bs = lax.rem(j, NBS)
            pltpu.make_async_copy(
                table_hbm.at[idx_v.at[j]], rows_v.at[bg], gsem.at[bg]
            ).wait()

            @pl.when(j >= NBS)
            def _():
                pltpu.make_async_copy(
                    cbuf.at[bs], out_slice(j - NBS), ssem.at[bs]
                ).wait()

            def compact(r, c):
                for g in range(D // 16):
                    cbuf[bs, r, pl.ds(g * 16, 16)] = rows_v[
                        bg, r, pl.ds(g * 16, 16)
                    ]
                return c

            lax.fori_loop(0, K, compact, 0)
            pltpu.async_copy(cbuf.at[bs], out_slice(j), ssem.at[bs])
            return carry

        lax.fori_loop(0, n_chunks, body, 0)

        for i in range(NBS):
            j = n_chunks - NBS + i
            pltpu.make_async_copy(
                cbuf.at[j % NBS], out_slice(j), ssem.at[j % NBS]
            ).wait()

    out = emb(idx3, table_p)
    return out.reshape(B, S, D)


# speedup vs baseline: 1.1722x; 1.1372x over previous
import functools

import jax
import jax.numpy as jnp
from jax import lax
from jax.experimental import pallas as pl
from jax.experimental.pallas import tpu as pltpu
from jax.experimental.pallas import tpu_sc as plsc

NC = 2
NS = 16
NW = NC * NS
NBLK = 32


def kernel(phone, table):
    B, S = phone.shape
    V, D = table.shape
    per_w = B // NW
    n_blk = per_w // NBLK
    idx3 = phone.reshape(NW, per_w, S)
    table_p = jnp.pad(table, ((0, 0), (0, 128 - D)))

    mesh = plsc.VectorSubcoreMesh(core_axis_name="c", subcore_axis_name="s")

    @functools.partial(
        pl.kernel,
        mesh=mesh,
        out_type=jax.ShapeDtypeStruct((B, S, D), jnp.float32),
        scratch_types=[
            pltpu.VMEM((2, NBLK, S), jnp.int32),
            pltpu.VMEM((2, S, 128), jnp.float32),
            pltpu.VMEM((2, S, D), jnp.float32),
            pltpu.SemaphoreType.DMA((2,)),
            pltpu.SemaphoreType.DMA((2,)),
            pltpu.SemaphoreType.DMA((2,)),
        ],
    )
    def emb(idx_hbm, table_hbm, out_hbm, ibuf, rows_v, cbuf, isem, gsem, ssem):
        wid = lax.axis_index("s") * NC + lax.axis_index("c")
        base = wid * per_w

        def idx_block(m):
            bm = m % 2
            return (
                idx_hbm.at[wid, pl.ds(m * NBLK, NBLK)],
                ibuf.at[bm],
                isem.at[bm],
            )

        def gather_parts(m, jj, b):
            bm = m % 2
            yield (
                table_hbm.at[ibuf.at[bm, jj, pl.ds(0, 128)]],
                rows_v.at[b, pl.ds(0, 128)],
            )
            yield (
                table_hbm.at[ibuf.at[bm, jj, pl.ds(128, S - 128)]],
                rows_v.at[b, pl.ds(128, S - 128)],
            )

        def fire_gather(m, jj, b):
            for src, dst in gather_parts(m, jj, b):
                pltpu.async_copy(src, dst, gsem.at[b])

        def wait_gather(m, jj, b):
            for src, dst in gather_parts(m, jj, b):
                pltpu.make_async_copy(src, dst, gsem.at[b]).wait()

        pltpu.async_copy(*idx_block(0))
        for m in range(n_blk):
            if m + 1 < n_blk:
                pltpu.async_copy(*idx_block(m + 1))
            pltpu.make_async_copy(*idx_block(m)).wait()
            fire_gather(m, 0, 0)

            def body(jj, carry):
                j = m * NBLK + jj

                @pl.when(jj + 1 < NBLK)
                def _():
                    fire_gather(m, jj + 1, lax.rem(jj + 1, 2))

                bg = lax.rem(jj, 2)
                bs = lax.rem(j, 2)
                wait_gather(m, jj, bg)

                @pl.when(j >= 2)
                def _():
                    pltpu.make_async_copy(
                        cbuf.at[bs], out_hbm.at[base + j - 2], ssem.at[bs]
                    ).wait()

                def compact(r, c):
                    for g in range(D // 16):
                        cbuf[bs, r, pl.ds(g * 16, 16)] = rows_v[
                            bg, r, pl.ds(g * 16, 16)
                        ]
                    return c

                lax.fori_loop(0, S, compact, 0)
                pltpu.async_copy(cbuf.at[bs], out_hbm.at[base + j], ssem.at[bs])
                return carry

            lax.fori_loop(0, NBLK, body, 0)

        for i in range(2):
            j = per_w - 2 + i
            pltpu.make_async_copy(
                cbuf.at[j % 2], out_hbm.at[base + j], ssem.at[j % 2]
            ).wait()

    return emb(idx3, table_p)
